# baseline (speedup 1.0000x reference)
import jax
import jax.numpy as jnp
from jax import lax
from jax.experimental import pallas as pl
from jax.experimental.pallas import tpu as pltpu
from jax.experimental.pallas import tpu_sc as plsc

_N = 10000
_D = 128
_E = 320000
_L = 16
_NC = 2
_NS = 16
_NW = _NC * _NS
_R = 10240
_CH = 128
_CPT = 80
_ACH = 64
_PK = 40
_EPT = _CPT * _CH
_EPAD = _NW * _EPT
_RPT = _R // _NS

_MESH = plsc.VectorSubcoreMesh(core_axis_name="c", subcore_axis_name="s")


def _ids():
    cid = lax.axis_index("c")
    sid = lax.axis_index("s")
    return cid, sid, sid * _NC + cid


def _fill_block(ref, rows, cols, val):
    def frow(i, carry):
        for j in range(cols // _L):
            ref[i, pl.ds(j * _L, _L)] = jnp.full((_L,), val, jnp.float32)
        return carry
    lax.fori_loop(0, rows, frow, 0)


def _count_body(dsts_hbm, cnt_hbm, dst_v, ones_v, cacc):
    cid, sid, wid = _ids()
    _fill_block(ones_v, _CH, _D, 0.0)
    base = sid * _RPT
    for k in range(_RPT // _CH):
        pltpu.sync_copy(ones_v, cacc.at[pl.ds(base + k * _CH, _CH)])
    _fill_block(ones_v, _CH, _D, 1.0)
    pltpu.sync_copy(dsts_hbm.at[wid], dst_v)
    plsc.subcore_barrier()

    def step(g, carry):
        pltpu.sync_copy(ones_v, cacc.at[dst_v.at[g]], add=True)
        return carry
    lax.fori_loop(0, _CPT, step, 0)
    plsc.subcore_barrier()
    for k in range(_RPT // _CH):
        sl = pl.ds(base + k * _CH, _CH)
        pltpu.sync_copy(cacc.at[sl], cnt_hbm.at[cid, sl])


_COUNT_SCRATCH = [
    pltpu.VMEM((_CPT, _CH), jnp.int32),
    pltpu.VMEM((_CH, _D), jnp.float32),
    pltpu.VMEM_SHARED((_R, _D), jnp.float32),
]

_COUNT = pl.kernel(
    _count_body,
    out_type=jax.ShapeDtypeStruct((_NC, _R, _D), jnp.float32),
    mesh=_MESH,
    scratch_types=_COUNT_SCRATCH,
)


def _agg_body(table_hbm, idx_hbm, out_hbm,
              idxbuf, buf, semi, sg0, sg1, sg2, sg3, acc):
    cid, sid, wid = _ids()

    _fill_block(buf.at[0], _ACH, _D, 0.0)
    base = sid * _RPT
    for k in range(_RPT // _ACH):
        pltpu.sync_copy(buf.at[0], acc.at[pl.ds(base + k * _ACH, _ACH)])
    plsc.subcore_barrier()

    sgs = (sg0, sg1, sg2, sg3)
    pltpu.async_copy(idx_hbm.at[wid, 0], idxbuf.at[0], semi)

    def step(g, carry):
        p = g % 2
        q = 1 - p
        pltpu.make_async_copy(idx_hbm.at[wid, g], idxbuf.at[p], semi).wait()
        ds = [pltpu.async_copy(table_hbm.at[idxbuf.at[p, i, 0]],
                               buf.at[i], sgs[i])
              for i in range(4)]

        @pl.when(g < _PK - 1)
        def _():
            pltpu.async_copy(idx_hbm.at[wid, g + 1], idxbuf.at[q], semi)

        for i in range(4):
            ds[i].wait()
            pltpu.sync_copy(buf.at[i], acc.at[idxbuf.at[p, i, 1]], add=True)
        return carry

    lax.fori_loop(0, _PK, step, 0)
    plsc.subcore_barrier()

    for k in range(_RPT // _ACH):
        sl = pl.ds(base + k * _ACH, _ACH)
        pltpu.sync_copy(acc.at[sl], out_hbm.at[cid, sl])


_AGG_SCRATCH = [
    pltpu.VMEM((2, 4, 2, _ACH), jnp.int32),
    pltpu.VMEM((4, _ACH, _D), jnp.float32),
    pltpu.SemaphoreType.DMA,
    pltpu.SemaphoreType.DMA,
    pltpu.SemaphoreType.DMA,
    pltpu.SemaphoreType.DMA,
    pltpu.SemaphoreType.DMA,
    pltpu.VMEM_SHARED((_R, _D), jnp.float32),
]

_AGG = pl.kernel(
    _agg_body,
    out_type=jax.ShapeDtypeStruct((_NC, _R, _D), jnp.float32),
    mesh=_MESH,
    scratch_types=_AGG_SCRATCH,
)

_BR = 512


def _make_tc_layer(relu_out):
    def body(s0, s1, c0, c1, x_r, wl, bl, wr, o):
        cnt = c0[:, 0:1] + c1[:, 0:1]
        inv = 1.0 / jnp.maximum(cnt, 1.0)
        mean = (s0[...] + s1[...]) * inv
        h = lax.dot_general(mean, wl[...], (((1,), (1,)), ((), ())),
                            precision=lax.Precision.HIGHEST,
                            preferred_element_type=jnp.float32)
        h = h + bl[...] + lax.dot_general(x_r[...], wr[...],
                                          (((1,), (1,)), ((), ())),
                                          precision=lax.Precision.HIGHEST,
                                          preferred_element_type=jnp.float32)
        if relu_out:
            h = jnp.maximum(h, 0.0)
        else:
            h = jnp.nan_to_num(h, nan=0.0, posinf=10000.0, neginf=-10000.0)
        o[...] = h

    row = lambda i: (i, 0)
    fixed = lambda i: (0, 0)
    return pl.pallas_call(
        body,
        grid=(_R // _BR,),
        in_specs=[
            pl.BlockSpec((_BR, _D), row),
            pl.BlockSpec((_BR, _D), row),
            pl.BlockSpec((_BR, _D), row),
            pl.BlockSpec((_BR, _D), row),
            pl.BlockSpec((_BR, _D), row),
            pl.BlockSpec((_D, _D), fixed),
            pl.BlockSpec((1, _D), fixed),
            pl.BlockSpec((_D, _D), fixed),
        ],
        out_specs=pl.BlockSpec((_BR, _D), row),
        out_shape=jax.ShapeDtypeStruct((_R, _D), jnp.float32),
    )


_TC_RELU = _make_tc_layer(True)
_TC_FINAL = _make_tc_layer(False)


def kernel(x, edge_index, W1_l, b1_l, W1_r, W2_l, b2_l, W2_r):
    src = edge_index[0].astype(jnp.int32)
    dst = edge_index[1].astype(jnp.int32)
    pad_dst = _N + jnp.arange(_EPAD - _E, dtype=jnp.int32) % (_R - _N)
    src = jnp.concatenate([src, jnp.zeros((_EPAD - _E,), jnp.int32)])
    dst = jnp.concatenate([dst, pad_dst])
    dsts = dst.reshape(_NW, _CPT, _CH)
    idxp = jnp.stack([src.reshape(_NW, _PK, 4, _ACH),
                      dst.reshape(_NW, _PK, 4, _ACH)], axis=3)
    x_pad = jnp.pad(x, ((0, _R - _N), (0, 0)))

    cnts = _COUNT(dsts)
    parts1 = _AGG(x_pad, idxp)
    h = _TC_RELU(parts1[0], parts1[1], cnts[0], cnts[1], x_pad,
                 W1_l, b1_l.reshape(1, _D), W1_r)
    parts2 = _AGG(h, idxp)
    out = _TC_FINAL(parts2[0], parts2[1], cnts[0], cnts[1], h,
                    W2_l, b2_l.reshape(1, _D), W2_r)
    return out[:_N]


# speedup vs baseline: 1.1741x; 1.1741x over previous
import jax
import jax.numpy as jnp
from jax import lax
from jax.experimental import pallas as pl
from jax.experimental.pallas import tpu as pltpu
from jax.experimental.pallas import tpu_sc as plsc

_N = 10000
_D = 128
_E = 320000
_L = 16
_NC = 2
_NS = 16
_NW = _NC * _NS
_R = 10240
_CH = 128
_CPT = 80
_EPT = _CPT * _CH
_EPAD = _NW * _EPT
_RPT = _R // _NS

_MESH = plsc.VectorSubcoreMesh(core_axis_name="c", subcore_axis_name="s")


def _ids():
    cid = lax.axis_index("c")
    sid = lax.axis_index("s")
    return cid, sid, sid * _NC + cid


def _fill_block(ref, rows, cols, val):
    def frow(i, carry):
        for j in range(cols // _L):
            ref[i, pl.ds(j * _L, _L)] = jnp.full((_L,), val, jnp.float32)
        return carry
    lax.fori_loop(0, rows, frow, 0)


def _count_body(dsts_hbm, cnt_hbm, dst_v, ones_v, cacc):
    cid, sid, wid = _ids()
    _fill_block(ones_v, _CH, _D, 0.0)
    base = sid * _RPT
    for k in range(_RPT // _CH):
        pltpu.sync_copy(ones_v, cacc.at[pl.ds(base + k * _CH, _CH)])
    _fill_block(ones_v, _CH, _D, 1.0)
    pltpu.sync_copy(dsts_hbm.at[wid], dst_v)
    plsc.subcore_barrier()

    def step(g, carry):
        pltpu.sync_copy(ones_v, cacc.at[dst_v.at[g]], add=True)
        return carry
    lax.fori_loop(0, _CPT, step, 0)
    plsc.subcore_barrier()
    for k in range(_RPT // _CH):
        sl = pl.ds(base + k * _CH, _CH)
        pltpu.sync_copy(cacc.at[sl], cnt_hbm.at[cid, sl])


_COUNT_SCRATCH = [
    pltpu.VMEM((_CPT, _CH), jnp.int32),
    pltpu.VMEM((_CH, _D), jnp.float32),
    pltpu.VMEM_SHARED((_R, _D), jnp.float32),
]

_COUNT = pl.kernel(
    _count_body,
    out_type=jax.ShapeDtypeStruct((_NC, _R, _D), jnp.float32),
    mesh=_MESH,
    scratch_types=_COUNT_SCRATCH,
)


def _agg_body(table_hbm, srcs_hbm, dsts_hbm, out_hbm,
              srca, srcb, dst_v, bufa, bufb, sia, sib, sga, sgb, acc):
    cid, sid, wid = _ids()

    _fill_block(bufa, _CH, _D, 0.0)
    base = sid * _RPT
    for k in range(_RPT // _CH):
        pltpu.sync_copy(bufa, acc.at[pl.ds(base + k * _CH, _CH)])
    pltpu.sync_copy(dsts_hbm.at[wid], dst_v)
    plsc.subcore_barrier()

    pltpu.async_copy(srcs_hbm.at[wid, 0], srca, sia)

    def step(g, carry):
        ia = 2 * g
        ib = 2 * g + 1
        pltpu.make_async_copy(srcs_hbm.at[wid, ia], srca, sia).wait()
        pltpu.async_copy(table_hbm.at[srca], bufa, sga)

        @pl.when(g > 0)
        def _():
            pltpu.make_async_copy(table_hbm.at[srcb], bufb, sgb).wait()
        pltpu.async_copy(srcs_hbm.at[wid, ib], srcb, sib)

        @pl.when(g > 0)
        def _():
            pltpu.sync_copy(bufb, acc.at[dst_v.at[ia - 1]], add=True)

        pltpu.make_async_copy(srcs_hbm.at[wid, ib], srcb, sib).wait()
        pltpu.async_copy(table_hbm.at[srcb], bufb, sgb)
        pltpu.make_async_copy(table_hbm.at[srca], bufa, sga).wait()

        @pl.when(g < _CPT // 2 - 1)
        def _():
            pltpu.async_copy(srcs_hbm.at[wid, ia + 2], srca, sia)

        pltpu.sync_copy(bufa, acc.at[dst_v.at[ia]], add=True)
        return carry

    lax.fori_loop(0, _CPT // 2, step, 0)
    pltpu.make_async_copy(table_hbm.at[srcb], bufb, sgb).wait()
    pltpu.sync_copy(bufb, acc.at[dst_v.at[_CPT - 1]], add=True)
    plsc.subcore_barrier()

    for k in range(_RPT // _CH):
        sl = pl.ds(base + k * _CH, _CH)
        pltpu.sync_copy(acc.at[sl], out_hbm.at[cid, sl])


_AGG_SCRATCH = [
    pltpu.VMEM((_CH,), jnp.int32),
    pltpu.VMEM((_CH,), jnp.int32),
    pltpu.VMEM((_CPT, _CH), jnp.int32),
    pltpu.VMEM((_CH, _D), jnp.float32),
    pltpu.VMEM((_CH, _D), jnp.float32),
    pltpu.SemaphoreType.DMA,
    pltpu.SemaphoreType.DMA,
    pltpu.SemaphoreType.DMA,
    pltpu.SemaphoreType.DMA,
    pltpu.VMEM_SHARED((_R, _D), jnp.float32),
]

_AGG = pl.kernel(
    _agg_body,
    out_type=jax.ShapeDtypeStruct((_NC, _R, _D), jnp.float32),
    mesh=_MESH,
    scratch_types=_AGG_SCRATCH,
)

_BR = 512


def _make_tc_layer(relu_out):
    def body(s0, s1, c0, c1, x_r, wl, bl, wr, o):
        cnt = c0[:, 0:1] + c1[:, 0:1]
        inv = 1.0 / jnp.maximum(cnt, 1.0)
        mean = (s0[...] + s1[...]) * inv
        h = lax.dot_general(mean, wl[...], (((1,), (1,)), ((), ())),
                            precision=lax.Precision.HIGHEST,
                            preferred_element_type=jnp.float32)
        h = h + bl[...] + lax.dot_general(x_r[...], wr[...],
                                          (((1,), (1,)), ((), ())),
                                          precision=lax.Precision.HIGHEST,
                                          preferred_element_type=jnp.float32)
        if relu_out:
            h = jnp.maximum(h, 0.0)
        else:
            h = jnp.nan_to_num(h, nan=0.0, posinf=10000.0, neginf=-10000.0)
        o[...] = h

    row = lambda i: (i, 0)
    fixed = lambda i: (0, 0)
    return pl.pallas_call(
        body,
        grid=(_R // _BR,),
        in_specs=[
            pl.BlockSpec((_BR, _D), row),
            pl.BlockSpec((_BR, _D), row),
            pl.BlockSpec((_BR, _D), row),
            pl.BlockSpec((_BR, _D), row),
            pl.BlockSpec((_BR, _D), row),
            pl.BlockSpec((_D, _D), fixed),
            pl.BlockSpec((1, _D), fixed),
            pl.BlockSpec((_D, _D), fixed),
        ],
        out_specs=pl.BlockSpec((_BR, _D), row),
        out_shape=jax.ShapeDtypeStruct((_R, _D), jnp.float32),
    )


_TC_RELU = _make_tc_layer(True)
_TC_FINAL = _make_tc_layer(False)


def kernel(x, edge_index, W1_l, b1_l, W1_r, W2_l, b2_l, W2_r):
    src = edge_index[0].astype(jnp.int32)
    dst = edge_index[1].astype(jnp.int32)
    pad_dst = _N + jnp.arange(_EPAD - _E, dtype=jnp.int32) % (_R - _N)
    src = jnp.concatenate([src, jnp.zeros((_EPAD - _E,), jnp.int32)])
    dst = jnp.concatenate([dst, pad_dst])
    srcs = src.reshape(_NW, _CPT, _CH)
    dsts = dst.reshape(_NW, _CPT, _CH)
    x_pad = jnp.pad(x, ((0, _R - _N), (0, 0)))

    cnts = _COUNT(dsts)
    parts1 = _AGG(x_pad, srcs, dsts)
    h = _TC_RELU(parts1[0], parts1[1], cnts[0], cnts[1], x_pad,
                 W1_l, b1_l.reshape(1, _D), W1_r)
    parts2 = _AGG(h, srcs, dsts)
    out = _TC_FINAL(parts2[0], parts2[1], cnts[0], cnts[1], h,
                    W2_l, b2_l.reshape(1, _D), W2_r)
    return out[:_N]
